# baseline (speedup 1.0000x reference)
import functools

import jax
import jax.numpy as jnp
from jax import lax
from jax.experimental import pallas as pl
from jax.experimental.pallas import tpu as pltpu
from jax.experimental.pallas import tpu_sc as plsc

K_BINS = 8192
EMB = 32
N_TOK = 32768
BT = 512
GRID = N_TOK // BT

SC_CORES = 2
SC_SUBCORES = 16
NW = SC_CORES * SC_SUBCORES
B_PER_W = N_TOK // NW
IDX_CHUNK = 128
N_CHUNK = B_PER_W // IDX_CHUNK


CHW = 128
N_CH = K_BINS // CHW


N_SUB = 4
SBT = BT // N_SUB


def _argmin_chain(xf, kw_ref, ksq):
    xsq = jnp.sum(xf * xf, axis=1, keepdims=True)

    def chunk_dist(c):
        mm = lax.dot_general(
            xf, kw_ref[:, c * CHW:(c + 1) * CHW], (((1,), (0,)), ((), ())),
            preferred_element_type=jnp.float32)
        return (xsq + mm) + ksq[:, c * CHW:(c + 1) * CHW]

    run_val = chunk_dist(0)
    run_ch = jnp.zeros(run_val.shape, jnp.int32)
    for c in range(1, N_CH):
        d = chunk_dist(c)
        lt = d < run_val
        run_val = jnp.where(lt, d, run_val)
        run_ch = jnp.where(lt, jnp.int32(c), run_ch)

    minv = jnp.min(run_val, axis=1, keepdims=True)
    lane = lax.broadcasted_iota(jnp.int32, run_val.shape, 1)
    cand = run_ch * CHW + lane
    idx = jnp.min(jnp.where(run_val == minv, cand, K_BINS),
                  axis=1, keepdims=True)
    return idx, minv, xsq


def _quant_body(xf_ref, kw_ref, ksq_ref, xl_ref, mind_ref, s1_ref, s2_ref):
    g = pl.program_id(0)
    ksq = ksq_ref[...]
    parts = [
        _argmin_chain(xf_ref[pl.ds(s * SBT, SBT), :], kw_ref, ksq)
        for s in range(N_SUB)
    ]
    xl_ref[...] = jnp.concatenate([p[0] for p in parts], axis=0)

    @pl.when(g == 0)
    def _():
        mind_ref[...] = jnp.zeros_like(mind_ref)
        s1_ref[...] = jnp.zeros_like(s1_ref)
        s2_ref[...] = jnp.zeros_like(s2_ref)

    mind_ref[...] += sum(jnp.sum(p[1]) for p in parts)
    s1_ref[...] += jnp.sum(xf_ref[...])
    s2_ref[...] += sum(jnp.sum(p[2]) for p in parts)


def _quantize(xf, kw, ksq):
    n_tok = xf.shape[0]
    return pl.pallas_call(
        _quant_body,
        grid=(n_tok // BT,),
        in_specs=[
            pl.BlockSpec((BT, EMB), lambda g: (g, 0)),
            pl.BlockSpec((EMB, K_BINS), lambda g: (0, 0)),
            pl.BlockSpec((1, K_BINS), lambda g: (0, 0)),
        ],
        out_specs=[
            pl.BlockSpec((BT, 1), lambda g: (g, 0)),
            pl.BlockSpec((1, 1), lambda g: (0, 0)),
            pl.BlockSpec((1, 1), lambda g: (0, 0)),
            pl.BlockSpec((1, 1), lambda g: (0, 0)),
        ],
        out_shape=[
            jax.ShapeDtypeStruct((n_tok, 1), jnp.int32),
            jax.ShapeDtypeStruct((1, 1), jnp.float32),
            jax.ShapeDtypeStruct((1, 1), jnp.float32),
            jax.ShapeDtypeStruct((1, 1), jnp.float32),
        ],
        compiler_params=pltpu.CompilerParams(
            dimension_semantics=("arbitrary",)),
    )(xf, kw, ksq)


def _dequant_sc(k, idx3):
    mesh = plsc.VectorSubcoreMesh(core_axis_name="c", subcore_axis_name="s")
    n_chunk = idx3.shape[1]
    b_per_w = n_chunk * IDX_CHUNK

    @functools.partial(
        pl.kernel,
        mesh=mesh,
        out_type=jax.ShapeDtypeStruct((NW, b_per_w, EMB), jnp.float32),
        scratch_types=[
            pltpu.VMEM((n_chunk, IDX_CHUNK), jnp.int32),
            pltpu.VMEM((b_per_w, EMB), jnp.float32),
            pltpu.SemaphoreType.DMA,
        ],
        compiler_params=pltpu.CompilerParams(use_tc_tiling_on_sc=False),
    )
    def gather_rows(k_hbm, idx_hbm, out_hbm, idx_v, rows_v, sem):
        wid = lax.axis_index("s") * SC_CORES + lax.axis_index("c")
        pltpu.sync_copy(idx_hbm.at[wid], idx_v)
        for j in range(n_chunk):
            pltpu.async_copy(
                k_hbm.at[idx_v.at[j]],
                rows_v.at[pl.ds(j * IDX_CHUNK, IDX_CHUNK)],
                sem,
            ).wait()
        pltpu.sync_copy(rows_v, out_hbm.at[wid])

    return gather_rows(k, idx3)


def kernel(x, k, update_k):
    N, width, T = x.shape
    xf = jnp.transpose(x, (0, 2, 1)).reshape(-1, width)
    kw = k.T
    ksq = jnp.sum(kw ** 2, axis=0, keepdims=True)
    kw2 = -2.0 * kw

    half = N_TOK // 2
    xl_a, mind_a, s1_a, s2_a = _quantize(xf[:half], kw2, ksq)
    rows_a = _dequant_sc(k, xl_a.reshape(NW, -1, IDX_CHUNK))
    xl_b, mind_b, s1_b, s2_b = _quantize(xf[half:], kw2, ksq)
    rows_b = _dequant_sc(k, xl_b.reshape(NW, -1, IDX_CHUNK))

    xl_flat = jnp.concatenate(
        [xl_a.reshape(half), xl_b.reshape(half)], axis=0)
    x_l = xl_flat.reshape(N, T)
    rows = jnp.concatenate(
        [rows_a.reshape(half, width), rows_b.reshape(half, width)], axis=0)
    x_d = jnp.transpose(rows.reshape(N, T, width), (0, 2, 1))

    n_el = jnp.float32(N_TOK * width)
    sum_min = mind_a[0, 0] + mind_b[0, 0]
    fit = sum_min / jnp.float32(N_TOK)
    commit_loss = sum_min / n_el
    s1v = s1_a[0, 0] + s1_b[0, 0]
    s2v = s2_a[0, 0] + s2_b[0, 0]
    prenorm = jnp.sqrt(jnp.maximum(s2v - s1v * s1v / n_el, 0.0) / n_el)

    return (x_l, x_d, commit_loss, fit, prenorm)


# speedup vs baseline: 1.0730x; 1.0730x over previous
import functools

import jax
import jax.numpy as jnp
from jax import lax
from jax.experimental import pallas as pl
from jax.experimental.pallas import tpu as pltpu
from jax.experimental.pallas import tpu_sc as plsc

K_BINS = 8192
EMB = 32
N_TOK = 32768
BT = 512
GRID = N_TOK // BT

SC_CORES = 2
SC_SUBCORES = 16
NW = SC_CORES * SC_SUBCORES
B_PER_W = N_TOK // NW
IDX_CHUNK = 128
N_CHUNK = B_PER_W // IDX_CHUNK


CHW = 128
N_CH = K_BINS // CHW


N_SUB = 4
SBT = BT // N_SUB


def _argmin_chain(xf, kw_ref, ksq):
    xsq = jnp.sum(xf * xf, axis=1, keepdims=True)

    def chunk_dist(c):
        mm = lax.dot_general(
            xf, kw_ref[:, c * CHW:(c + 1) * CHW], (((1,), (0,)), ((), ())),
            preferred_element_type=jnp.float32)
        return (xsq + mm) + ksq[:, c * CHW:(c + 1) * CHW]

    run_val = chunk_dist(0)
    run_ch = jnp.zeros(run_val.shape, jnp.int32)
    for c in range(1, N_CH):
        d = chunk_dist(c)
        lt = d < run_val
        run_val = jnp.where(lt, d, run_val)
        run_ch = jnp.where(lt, jnp.int32(c), run_ch)

    minv = jnp.min(run_val, axis=1, keepdims=True)
    lane = lax.broadcasted_iota(jnp.int32, run_val.shape, 1)
    cand = run_ch * CHW + lane
    idx = jnp.min(jnp.where(run_val == minv, cand, K_BINS),
                  axis=1, keepdims=True)
    return idx, minv, xsq


def _quant_body(xf_ref, kw_ref, ksq_ref, xl_ref, mind_ref, s1_ref, s2_ref):
    g = pl.program_id(0)
    ksq = ksq_ref[...]
    parts = [
        _argmin_chain(xf_ref[pl.ds(s * SBT, SBT), :], kw_ref, ksq)
        for s in range(N_SUB)
    ]
    xl_ref[...] = jnp.concatenate([p[0] for p in parts], axis=0)

    @pl.when(g == 0)
    def _():
        mind_ref[...] = jnp.zeros_like(mind_ref)
        s1_ref[...] = jnp.zeros_like(s1_ref)
        s2_ref[...] = jnp.zeros_like(s2_ref)

    mind_ref[...] += sum(jnp.sum(p[1]) for p in parts)
    s1_ref[...] += jnp.sum(xf_ref[...])
    s2_ref[...] += sum(jnp.sum(p[2]) for p in parts)


def _quantize(xf, kw, ksq):
    n_tok = xf.shape[0]
    return pl.pallas_call(
        _quant_body,
        grid=(n_tok // BT,),
        in_specs=[
            pl.BlockSpec((BT, EMB), lambda g: (g, 0)),
            pl.BlockSpec((EMB, K_BINS), lambda g: (0, 0)),
            pl.BlockSpec((1, K_BINS), lambda g: (0, 0)),
        ],
        out_specs=[
            pl.BlockSpec((BT, 1), lambda g: (g, 0)),
            pl.BlockSpec((1, 1), lambda g: (0, 0)),
            pl.BlockSpec((1, 1), lambda g: (0, 0)),
            pl.BlockSpec((1, 1), lambda g: (0, 0)),
        ],
        out_shape=[
            jax.ShapeDtypeStruct((n_tok, 1), jnp.int32),
            jax.ShapeDtypeStruct((1, 1), jnp.float32),
            jax.ShapeDtypeStruct((1, 1), jnp.float32),
            jax.ShapeDtypeStruct((1, 1), jnp.float32),
        ],
        compiler_params=pltpu.CompilerParams(
            dimension_semantics=("arbitrary",)),
    )(xf, kw, ksq)


def _dequant_sc(k, idx3):
    mesh = plsc.VectorSubcoreMesh(core_axis_name="c", subcore_axis_name="s")
    n_chunk = idx3.shape[1]
    b_per_w = n_chunk * IDX_CHUNK

    @functools.partial(
        pl.kernel,
        mesh=mesh,
        out_type=jax.ShapeDtypeStruct((NW, b_per_w, EMB), jnp.float32),
        scratch_types=[
            pltpu.VMEM((n_chunk, IDX_CHUNK), jnp.int32),
            pltpu.VMEM((b_per_w, EMB), jnp.float32),
            pltpu.SemaphoreType.DMA,
        ],
        compiler_params=pltpu.CompilerParams(use_tc_tiling_on_sc=False),
    )
    def gather_rows(k_hbm, idx_hbm, out_hbm, idx_v, rows_v, sem):
        wid = lax.axis_index("s") * SC_CORES + lax.axis_index("c")
        pltpu.sync_copy(idx_hbm.at[wid], idx_v)
        for j in range(n_chunk):
            pltpu.async_copy(
                k_hbm.at[idx_v.at[j]],
                rows_v.at[pl.ds(j * IDX_CHUNK, IDX_CHUNK)],
                sem,
            ).wait()
        pltpu.sync_copy(rows_v, out_hbm.at[wid])

    return gather_rows(k, idx3)


def kernel(x, k, update_k):
    N, width, T = x.shape
    xf = jnp.transpose(x, (0, 2, 1)).reshape(-1, width)
    kw = k.T
    ksq = jnp.sum(kw ** 2, axis=0, keepdims=True)
    kw2 = -2.0 * kw

    xl_col, mind, s1, s2 = _quantize(xf, kw2, ksq)

    xl_flat = xl_col.reshape(N_TOK)
    x_l = xl_flat.reshape(N, T)

    idx3 = xl_flat.reshape(NW, N_CHUNK, IDX_CHUNK)
    rows = _dequant_sc(k, idx3)
    x_d = jnp.transpose(rows.reshape(N, T, width), (0, 2, 1))

    n_el = jnp.float32(N_TOK * width)
    sum_min = mind[0, 0]
    fit = sum_min / jnp.float32(N_TOK)
    commit_loss = sum_min / n_el
    s1v, s2v = s1[0, 0], s2[0, 0]
    prenorm = jnp.sqrt(jnp.maximum(s2v - s1v * s1v / n_el, 0.0) / n_el)

    return (x_l, x_d, commit_loss, fit, prenorm)


# speedup vs baseline: 1.1012x; 1.0263x over previous
import functools

import jax
import jax.numpy as jnp
from jax import lax
from jax.experimental import pallas as pl
from jax.experimental.pallas import tpu as pltpu
from jax.experimental.pallas import tpu_sc as plsc

K_BINS = 8192
EMB = 32
N_TOK = 32768
BT = 512
GRID = N_TOK // BT

SC_CORES = 2
SC_SUBCORES = 16
NW = SC_CORES * SC_SUBCORES
B_PER_W = N_TOK // NW
IDX_CHUNK = 128
N_CHUNK = B_PER_W // IDX_CHUNK


CHW = 128
N_CH = K_BINS // CHW


N_SUB = 8
SBT = BT // N_SUB


def _argmin_chain(xf, kw_ref, ksq):
    xsq = jnp.sum(xf * xf, axis=1, keepdims=True)

    def chunk_dist(c):
        mm = lax.dot_general(
            xf, kw_ref[:, c * CHW:(c + 1) * CHW], (((1,), (0,)), ((), ())),
            preferred_element_type=jnp.float32)
        return (xsq + mm) + ksq[:, c * CHW:(c + 1) * CHW]

    run_val = chunk_dist(0)
    run_ch = jnp.zeros(run_val.shape, jnp.int32)
    for c in range(1, N_CH):
        d = chunk_dist(c)
        lt = d < run_val
        run_val = jnp.where(lt, d, run_val)
        run_ch = jnp.where(lt, jnp.int32(c), run_ch)

    minv = jnp.min(run_val, axis=1, keepdims=True)
    lane = lax.broadcasted_iota(jnp.int32, run_val.shape, 1)
    cand = run_ch * CHW + lane
    idx = jnp.min(jnp.where(run_val == minv, cand, K_BINS),
                  axis=1, keepdims=True)
    return idx, minv, xsq


def _quant_body(xf_ref, kw_ref, ksq_ref, xl_ref, mind_ref, s1_ref, s2_ref):
    g = pl.program_id(0)
    ksq = ksq_ref[...]
    parts = [
        _argmin_chain(xf_ref[pl.ds(s * SBT, SBT), :], kw_ref, ksq)
        for s in range(N_SUB)
    ]
    xl_ref[...] = jnp.concatenate([p[0] for p in parts], axis=0)

    @pl.when(g == 0)
    def _():
        mind_ref[...] = jnp.zeros_like(mind_ref)
        s1_ref[...] = jnp.zeros_like(s1_ref)
        s2_ref[...] = jnp.zeros_like(s2_ref)

    mind_ref[...] += sum(jnp.sum(p[1]) for p in parts)
    s1_ref[...] += jnp.sum(xf_ref[...])
    s2_ref[...] += sum(jnp.sum(p[2]) for p in parts)


def _quantize(xf, kw, ksq):
    n_tok = xf.shape[0]
    return pl.pallas_call(
        _quant_body,
        grid=(n_tok // BT,),
        in_specs=[
            pl.BlockSpec((BT, EMB), lambda g: (g, 0)),
            pl.BlockSpec((EMB, K_BINS), lambda g: (0, 0)),
            pl.BlockSpec((1, K_BINS), lambda g: (0, 0)),
        ],
        out_specs=[
            pl.BlockSpec((BT, 1), lambda g: (g, 0)),
            pl.BlockSpec((1, 1), lambda g: (0, 0)),
            pl.BlockSpec((1, 1), lambda g: (0, 0)),
            pl.BlockSpec((1, 1), lambda g: (0, 0)),
        ],
        out_shape=[
            jax.ShapeDtypeStruct((n_tok, 1), jnp.int32),
            jax.ShapeDtypeStruct((1, 1), jnp.float32),
            jax.ShapeDtypeStruct((1, 1), jnp.float32),
            jax.ShapeDtypeStruct((1, 1), jnp.float32),
        ],
        compiler_params=pltpu.CompilerParams(
            dimension_semantics=("arbitrary",)),
    )(xf, kw, ksq)


def _dequant_sc(k, idx3):
    mesh = plsc.VectorSubcoreMesh(core_axis_name="c", subcore_axis_name="s")
    n_chunk = idx3.shape[1]
    b_per_w = n_chunk * IDX_CHUNK

    @functools.partial(
        pl.kernel,
        mesh=mesh,
        out_type=jax.ShapeDtypeStruct((NW, b_per_w, EMB), jnp.float32),
        scratch_types=[
            pltpu.VMEM((n_chunk, IDX_CHUNK), jnp.int32),
            pltpu.VMEM((b_per_w, EMB), jnp.float32),
            pltpu.SemaphoreType.DMA,
        ],
        compiler_params=pltpu.CompilerParams(use_tc_tiling_on_sc=False),
    )
    def gather_rows(k_hbm, idx_hbm, out_hbm, idx_v, rows_v, sem):
        wid = lax.axis_index("s") * SC_CORES + lax.axis_index("c")
        pltpu.sync_copy(idx_hbm.at[wid], idx_v)
        for j in range(n_chunk):
            pltpu.async_copy(
                k_hbm.at[idx_v.at[j]],
                rows_v.at[pl.ds(j * IDX_CHUNK, IDX_CHUNK)],
                sem,
            ).wait()
        pltpu.sync_copy(rows_v, out_hbm.at[wid])

    return gather_rows(k, idx3)


def kernel(x, k, update_k):
    N, width, T = x.shape
    xf = jnp.transpose(x, (0, 2, 1)).reshape(-1, width)
    kw = k.T
    ksq = jnp.sum(kw ** 2, axis=0, keepdims=True)
    kw2 = -2.0 * kw

    xl_col, mind, s1, s2 = _quantize(xf, kw2, ksq)

    xl_flat = xl_col.reshape(N_TOK)
    x_l = xl_flat.reshape(N, T)

    idx3 = xl_flat.reshape(NW, N_CHUNK, IDX_CHUNK)
    rows = _dequant_sc(k, idx3)
    x_d = jnp.transpose(rows.reshape(N, T, width), (0, 2, 1))

    n_el = jnp.float32(N_TOK * width)
    sum_min = mind[0, 0]
    fit = sum_min / jnp.float32(N_TOK)
    commit_loss = sum_min / n_el
    s1v, s2v = s1[0, 0], s2[0, 0]
    prenorm = jnp.sqrt(jnp.maximum(s2v - s1v * s1v / n_el, 0.0) / n_el)

    return (x_l, x_d, commit_loss, fit, prenorm)


# speedup vs baseline: 1.1579x; 1.0515x over previous
import functools

import jax
import jax.numpy as jnp
from jax import lax
from jax.experimental import pallas as pl
from jax.experimental.pallas import tpu as pltpu
from jax.experimental.pallas import tpu_sc as plsc

K_BINS = 8192
EMB = 32
N_TOK = 32768
BT = 1024
GRID = N_TOK // BT

SC_CORES = 2
SC_SUBCORES = 16
NW = SC_CORES * SC_SUBCORES
B_PER_W = N_TOK // NW
IDX_CHUNK = 128
N_CHUNK = B_PER_W // IDX_CHUNK


CHW = 128
N_CH = K_BINS // CHW


N_SUB = 16
SBT = BT // N_SUB


def _argmin_chain(xf, kw_ref, ksq):
    xsq = jnp.sum(xf * xf, axis=1, keepdims=True)

    def chunk_dist(c):
        mm = lax.dot_general(
            xf, kw_ref[:, c * CHW:(c + 1) * CHW], (((1,), (0,)), ((), ())),
            preferred_element_type=jnp.float32)
        return (xsq + mm) + ksq[:, c * CHW:(c + 1) * CHW]

    run_val = chunk_dist(0)
    run_ch = jnp.zeros(run_val.shape, jnp.int32)
    for c in range(1, N_CH):
        d = chunk_dist(c)
        lt = d < run_val
        run_val = jnp.where(lt, d, run_val)
        run_ch = jnp.where(lt, jnp.int32(c), run_ch)

    minv = jnp.min(run_val, axis=1, keepdims=True)
    lane = lax.broadcasted_iota(jnp.int32, run_val.shape, 1)
    cand = run_ch * CHW + lane
    idx = jnp.min(jnp.where(run_val == minv, cand, K_BINS),
                  axis=1, keepdims=True)
    return idx, minv, xsq


def _quant_body(xf_ref, kw_ref, ksq_ref, xl_ref, mind_ref, s1_ref, s2_ref):
    g = pl.program_id(0)
    ksq = ksq_ref[...]
    parts = [
        _argmin_chain(xf_ref[pl.ds(s * SBT, SBT), :], kw_ref, ksq)
        for s in range(N_SUB)
    ]
    xl_ref[...] = jnp.concatenate([p[0] for p in parts], axis=0)

    @pl.when(g == 0)
    def _():
        mind_ref[...] = jnp.zeros_like(mind_ref)
        s1_ref[...] = jnp.zeros_like(s1_ref)
        s2_ref[...] = jnp.zeros_like(s2_ref)

    mind_ref[...] += sum(jnp.sum(p[1]) for p in parts)
    s1_ref[...] += jnp.sum(xf_ref[...])
    s2_ref[...] += sum(jnp.sum(p[2]) for p in parts)


def _quantize(xf, kw, ksq):
    n_tok = xf.shape[0]
    return pl.pallas_call(
        _quant_body,
        grid=(n_tok // BT,),
        in_specs=[
            pl.BlockSpec((BT, EMB), lambda g: (g, 0)),
            pl.BlockSpec((EMB, K_BINS), lambda g: (0, 0)),
            pl.BlockSpec((1, K_BINS), lambda g: (0, 0)),
        ],
        out_specs=[
            pl.BlockSpec((BT, 1), lambda g: (g, 0)),
            pl.BlockSpec((1, 1), lambda g: (0, 0)),
            pl.BlockSpec((1, 1), lambda g: (0, 0)),
            pl.BlockSpec((1, 1), lambda g: (0, 0)),
        ],
        out_shape=[
            jax.ShapeDtypeStruct((n_tok, 1), jnp.int32),
            jax.ShapeDtypeStruct((1, 1), jnp.float32),
            jax.ShapeDtypeStruct((1, 1), jnp.float32),
            jax.ShapeDtypeStruct((1, 1), jnp.float32),
        ],
        compiler_params=pltpu.CompilerParams(
            dimension_semantics=("arbitrary",)),
    )(xf, kw, ksq)


def _dequant_sc(k, idx3):
    mesh = plsc.VectorSubcoreMesh(core_axis_name="c", subcore_axis_name="s")
    n_chunk = idx3.shape[1]
    b_per_w = n_chunk * IDX_CHUNK

    @functools.partial(
        pl.kernel,
        mesh=mesh,
        out_type=jax.ShapeDtypeStruct((NW, b_per_w, EMB), jnp.float32),
        scratch_types=[
            pltpu.VMEM((n_chunk, IDX_CHUNK), jnp.int32),
            pltpu.VMEM((b_per_w, EMB), jnp.float32),
            pltpu.SemaphoreType.DMA,
        ],
        compiler_params=pltpu.CompilerParams(use_tc_tiling_on_sc=False),
    )
    def gather_rows(k_hbm, idx_hbm, out_hbm, idx_v, rows_v, sem):
        wid = lax.axis_index("s") * SC_CORES + lax.axis_index("c")
        pltpu.sync_copy(idx_hbm.at[wid], idx_v)
        for j in range(n_chunk):
            pltpu.async_copy(
                k_hbm.at[idx_v.at[j]],
                rows_v.at[pl.ds(j * IDX_CHUNK, IDX_CHUNK)],
                sem,
            ).wait()
        pltpu.sync_copy(rows_v, out_hbm.at[wid])

    return gather_rows(k, idx3)


def kernel(x, k, update_k):
    N, width, T = x.shape
    xf = jnp.transpose(x, (0, 2, 1)).reshape(-1, width)
    kw = k.T
    ksq = jnp.sum(kw ** 2, axis=0, keepdims=True)
    kw2 = -2.0 * kw

    xl_col, mind, s1, s2 = _quantize(xf, kw2, ksq)

    xl_flat = xl_col.reshape(N_TOK)
    x_l = xl_flat.reshape(N, T)

    idx3 = xl_flat.reshape(NW, N_CHUNK, IDX_CHUNK)
    rows = _dequant_sc(k, idx3)
    x_d = jnp.transpose(rows.reshape(N, T, width), (0, 2, 1))

    n_el = jnp.float32(N_TOK * width)
    sum_min = mind[0, 0]
    fit = sum_min / jnp.float32(N_TOK)
    commit_loss = sum_min / n_el
    s1v, s2v = s1[0, 0], s2[0, 0]
    prenorm = jnp.sqrt(jnp.maximum(s2v - s1v * s1v / n_el, 0.0) / n_el)

    return (x_l, x_d, commit_loss, fit, prenorm)
